# baseline (speedup 1.0000x reference)
import functools

import jax
import jax.numpy as jnp
from jax import lax
from jax.experimental import pallas as pl
from jax.experimental.pallas import tpu as pltpu
from jax.experimental.pallas import tpu_sc as plsc

_D = 4 * 64 * 64
_B = 1024
_CHUNKS = (4096, 8192, 4096)
_NC = 2
_NS = 16
_NW = _NC * _NS
_BPW = _B // _NW
_CH = 4
_NCH = _BPW // _CH
_DBLK = 512

_mesh = plsc.VectorSubcoreMesh(core_axis_name="c", subcore_axis_name="s")


def _make_chunk_kernel(d0, dc):
    @functools.partial(
        pl.kernel,
        mesh=_mesh,
        out_type=jax.ShapeDtypeStruct((_B, dc), jnp.float32),
        scratch_types=[
            pltpu.VMEM((_NCH, _CH), jnp.int32),
            pltpu.VMEM((2, _CH, dc), jnp.float32),
            pltpu.SemaphoreType.DMA,
            pltpu.SemaphoreType.DMA,
            pltpu.SemaphoreType.DMA,
            pltpu.SemaphoreType.DMA,
        ],
    )
    def _emb_gather(idx_hbm, table_hbm, out_hbm, idx_v, rows_v,
                    s_in0, s_in1, s_out0, s_out1):
        wid = lax.axis_index("s") * _NC + lax.axis_index("c")
        base = wid * _BPW
        pltpu.sync_copy(idx_hbm.at[wid], idx_v)
        s_in = (s_in0, s_in1)
        s_out = (s_out0, s_out1)

        def gather(c):
            b = c % 2
            return pltpu.make_async_copy(
                table_hbm.at[idx_v.at[c], pl.ds(d0, dc)], rows_v.at[b], s_in[b])

        def put(c):
            b = c % 2
            return pltpu.make_async_copy(
                rows_v.at[b], out_hbm.at[pl.ds(base + c * _CH, _CH)], s_out[b])

        gather(0).start()
        if _NCH > 1:
            gather(1).start()
        for c in range(_NCH):
            gather(c).wait()
            put(c).start()
            if c + 2 < _NCH:
                put(c).wait()
                gather(c + 2).start()
        if _NCH > 1:
            put(_NCH - 2).wait()
        put(_NCH - 1).wait()

    return _emb_gather


def _tp_body_first(chunk_ref, out_ref):
    out_ref[...] = chunk_ref[...].T


def _tp_body(chunk_ref, _buf_ref, out_ref):
    out_ref[...] = chunk_ref[...].T


def _make_transpose(d0, dc, aliased):
    grid = dc // _DBLK
    blk0 = d0 // _DBLK
    in_specs = [pl.BlockSpec((_B, _DBLK), lambda i: (0, i))]
    if aliased:
        in_specs.append(pl.BlockSpec(memory_space=pl.ANY))
    out_spec = pl.BlockSpec((_DBLK, _B), lambda i: (blk0 + i, 0))
    return pl.pallas_call(
        _tp_body if aliased else _tp_body_first,
        grid=(grid,),
        in_specs=in_specs,
        out_specs=out_spec,
        out_shape=jax.ShapeDtypeStruct((_D, _B), jnp.float32),
        input_output_aliases={1: 0} if aliased else {},
    )


_offsets = [sum(_CHUNKS[:k]) for k in range(len(_CHUNKS))]
_chunk_kernels = [_make_chunk_kernel(_offsets[k], _CHUNKS[k])
                  for k in range(len(_CHUNKS))]
_transpose_kernels = [_make_transpose(_offsets[k], _CHUNKS[k], aliased=(k > 0))
                      for k in range(len(_CHUNKS))]


def kernel(x, table):
    idx = x.astype(jnp.int32).reshape(_NW, _NCH, _CH)
    buf = None
    for k in range(len(_CHUNKS)):
        ok = _chunk_kernels[k](idx, table)
        buf = _transpose_kernels[k](ok) if k == 0 else _transpose_kernels[k](ok, buf)
    out_t = buf.reshape(4, 64, 64, _B)
    return out_t.transpose(3, 0, 1, 2)


# speedup vs baseline: 1.0237x; 1.0237x over previous
import functools

import jax
import jax.numpy as jnp
from jax import lax
from jax.experimental import pallas as pl
from jax.experimental.pallas import tpu as pltpu
from jax.experimental.pallas import tpu_sc as plsc

_D = 4 * 64 * 64
_B = 1024
_CHUNKS = (8192, 8192)
_NC = 2
_NS = 16
_NW = _NC * _NS
_BPW = _B // _NW
_CH = 4
_NCH = _BPW // _CH
_DBLK = 512

_mesh = plsc.VectorSubcoreMesh(core_axis_name="c", subcore_axis_name="s")


def _make_chunk_kernel(d0, dc):
    @functools.partial(
        pl.kernel,
        mesh=_mesh,
        out_type=jax.ShapeDtypeStruct((_B, dc), jnp.float32),
        scratch_types=[
            pltpu.VMEM((_NCH, _CH), jnp.int32),
            pltpu.VMEM((2, _CH, dc), jnp.float32),
            pltpu.SemaphoreType.DMA,
            pltpu.SemaphoreType.DMA,
            pltpu.SemaphoreType.DMA,
            pltpu.SemaphoreType.DMA,
        ],
    )
    def _emb_gather(idx_hbm, table_hbm, out_hbm, idx_v, rows_v,
                    s_in0, s_in1, s_out0, s_out1):
        wid = lax.axis_index("s") * _NC + lax.axis_index("c")
        base = wid * _BPW
        pltpu.sync_copy(idx_hbm.at[wid], idx_v)
        s_in = (s_in0, s_in1)
        s_out = (s_out0, s_out1)

        def gather(c):
            b = c % 2
            return pltpu.make_async_copy(
                table_hbm.at[idx_v.at[c], pl.ds(d0, dc)], rows_v.at[b], s_in[b])

        def put(c):
            b = c % 2
            return pltpu.make_async_copy(
                rows_v.at[b], out_hbm.at[pl.ds(base + c * _CH, _CH)], s_out[b])

        gather(0).start()
        if _NCH > 1:
            gather(1).start()
        for c in range(_NCH):
            gather(c).wait()
            put(c).start()
            if c + 2 < _NCH:
                put(c).wait()
                gather(c + 2).start()
        if _NCH > 1:
            put(_NCH - 2).wait()
        put(_NCH - 1).wait()

    return _emb_gather


def _tp_body_first(chunk_ref, out_ref):
    out_ref[...] = chunk_ref[...].T


def _tp_body(chunk_ref, _buf_ref, out_ref):
    out_ref[...] = chunk_ref[...].T


def _make_transpose(d0, dc, aliased):
    grid = dc // _DBLK
    blk0 = d0 // _DBLK
    in_specs = [pl.BlockSpec((_B, _DBLK), lambda i: (0, i))]
    if aliased:
        in_specs.append(pl.BlockSpec(memory_space=pl.ANY))
    out_spec = pl.BlockSpec((_DBLK, _B), lambda i: (blk0 + i, 0))
    return pl.pallas_call(
        _tp_body if aliased else _tp_body_first,
        grid=(grid,),
        in_specs=in_specs,
        out_specs=out_spec,
        out_shape=jax.ShapeDtypeStruct((_D, _B), jnp.float32),
        input_output_aliases={1: 0} if aliased else {},
    )


_offsets = [sum(_CHUNKS[:k]) for k in range(len(_CHUNKS))]
_chunk_kernels = [_make_chunk_kernel(_offsets[k], _CHUNKS[k])
                  for k in range(len(_CHUNKS))]
_transpose_kernels = [_make_transpose(_offsets[k], _CHUNKS[k], aliased=(k > 0))
                      for k in range(len(_CHUNKS))]


def kernel(x, table):
    idx = x.astype(jnp.int32).reshape(_NW, _NCH, _CH)
    buf = None
    for k in range(len(_CHUNKS)):
        ok = _chunk_kernels[k](idx, table)
        buf = _transpose_kernels[k](ok) if k == 0 else _transpose_kernels[k](ok, buf)
    out_t = buf.reshape(4, 64, 64, _B)
    return out_t.transpose(3, 0, 1, 2)


# speedup vs baseline: 1.0451x; 1.0210x over previous
import functools

import jax
import jax.numpy as jnp
from jax import lax
from jax.experimental import pallas as pl
from jax.experimental.pallas import tpu as pltpu
from jax.experimental.pallas import tpu_sc as plsc

_D = 4 * 64 * 64
_B = 1024
_CHUNKS = (8192, 8192)
_NC = 2
_NS = 16
_NW = _NC * _NS
_BPW = _B // _NW
_CH = 4
_NCH = _BPW // _CH
_DBLK = 1024

_mesh = plsc.VectorSubcoreMesh(core_axis_name="c", subcore_axis_name="s")


def _make_chunk_kernel(d0, dc):
    @functools.partial(
        pl.kernel,
        mesh=_mesh,
        out_type=jax.ShapeDtypeStruct((_B, dc), jnp.float32),
        scratch_types=[
            pltpu.VMEM((_NCH, _CH), jnp.int32),
            pltpu.VMEM((2, _CH, dc), jnp.float32),
            pltpu.SemaphoreType.DMA,
            pltpu.SemaphoreType.DMA,
            pltpu.SemaphoreType.DMA,
            pltpu.SemaphoreType.DMA,
        ],
    )
    def _emb_gather(idx_hbm, table_hbm, out_hbm, idx_v, rows_v,
                    s_in0, s_in1, s_out0, s_out1):
        wid = lax.axis_index("s") * _NC + lax.axis_index("c")
        base = wid * _BPW
        pltpu.sync_copy(idx_hbm.at[wid], idx_v)
        s_in = (s_in0, s_in1)
        s_out = (s_out0, s_out1)

        def gather(c):
            b = c % 2
            return pltpu.make_async_copy(
                table_hbm.at[idx_v.at[c], pl.ds(d0, dc)], rows_v.at[b], s_in[b])

        def put(c):
            b = c % 2
            return pltpu.make_async_copy(
                rows_v.at[b], out_hbm.at[pl.ds(base + c * _CH, _CH)], s_out[b])

        gather(0).start()
        if _NCH > 1:
            gather(1).start()
        for c in range(_NCH):
            gather(c).wait()
            put(c).start()
            if c + 2 < _NCH:
                put(c).wait()
                gather(c + 2).start()
        if _NCH > 1:
            put(_NCH - 2).wait()
        put(_NCH - 1).wait()

    return _emb_gather


def _tp_body_first(chunk_ref, out_ref):
    out_ref[...] = chunk_ref[...].T


def _tp_body(chunk_ref, _buf_ref, out_ref):
    out_ref[...] = chunk_ref[...].T


def _make_transpose(d0, dc, aliased):
    grid = dc // _DBLK
    blk0 = d0 // _DBLK
    in_specs = [pl.BlockSpec((_B, _DBLK), lambda i: (0, i))]
    if aliased:
        in_specs.append(pl.BlockSpec(memory_space=pl.ANY))
    out_spec = pl.BlockSpec((_DBLK, _B), lambda i: (blk0 + i, 0))
    return pl.pallas_call(
        _tp_body if aliased else _tp_body_first,
        grid=(grid,),
        in_specs=in_specs,
        out_specs=out_spec,
        out_shape=jax.ShapeDtypeStruct((_D, _B), jnp.float32),
        input_output_aliases={1: 0} if aliased else {},
    )


_offsets = [sum(_CHUNKS[:k]) for k in range(len(_CHUNKS))]
_chunk_kernels = [_make_chunk_kernel(_offsets[k], _CHUNKS[k])
                  for k in range(len(_CHUNKS))]
_transpose_kernels = [_make_transpose(_offsets[k], _CHUNKS[k], aliased=(k > 0))
                      for k in range(len(_CHUNKS))]


def kernel(x, table):
    idx = x.astype(jnp.int32).reshape(_NW, _NCH, _CH)
    buf = None
    for k in range(len(_CHUNKS)):
        ok = _chunk_kernels[k](idx, table)
        buf = _transpose_kernels[k](ok) if k == 0 else _transpose_kernels[k](ok, buf)
    out_t = buf.reshape(4, 64, 64, _B)
    return out_t.transpose(3, 0, 1, 2)


# speedup vs baseline: 1.0550x; 1.0095x over previous
import functools

import jax
import jax.numpy as jnp
from jax import lax
from jax.experimental import pallas as pl
from jax.experimental.pallas import tpu as pltpu
from jax.experimental.pallas import tpu_sc as plsc

_D = 4 * 64 * 64
_B = 1024
_CHUNKS = (8192, 8192)
_NC = 2
_NS = 16
_NW = _NC * _NS
_BPW = _B // _NW
_CH = 4
_NCH = _BPW // _CH
_DBLK = 2048

_mesh = plsc.VectorSubcoreMesh(core_axis_name="c", subcore_axis_name="s")


def _make_chunk_kernel(d0, dc):
    @functools.partial(
        pl.kernel,
        mesh=_mesh,
        out_type=jax.ShapeDtypeStruct((_B, dc), jnp.float32),
        scratch_types=[
            pltpu.VMEM((_NCH, _CH), jnp.int32),
            pltpu.VMEM((2, _CH, dc), jnp.float32),
            pltpu.SemaphoreType.DMA,
            pltpu.SemaphoreType.DMA,
            pltpu.SemaphoreType.DMA,
            pltpu.SemaphoreType.DMA,
        ],
    )
    def _emb_gather(idx_hbm, table_hbm, out_hbm, idx_v, rows_v,
                    s_in0, s_in1, s_out0, s_out1):
        wid = lax.axis_index("s") * _NC + lax.axis_index("c")
        base = wid * _BPW
        pltpu.sync_copy(idx_hbm.at[wid], idx_v)
        s_in = (s_in0, s_in1)
        s_out = (s_out0, s_out1)

        def gather(c):
            b = c % 2
            return pltpu.make_async_copy(
                table_hbm.at[idx_v.at[c], pl.ds(d0, dc)], rows_v.at[b], s_in[b])

        def put(c):
            b = c % 2
            return pltpu.make_async_copy(
                rows_v.at[b], out_hbm.at[pl.ds(base + c * _CH, _CH)], s_out[b])

        gather(0).start()
        if _NCH > 1:
            gather(1).start()
        for c in range(_NCH):
            gather(c).wait()
            put(c).start()
            if c + 2 < _NCH:
                put(c).wait()
                gather(c + 2).start()
        if _NCH > 1:
            put(_NCH - 2).wait()
        put(_NCH - 1).wait()

    return _emb_gather


def _tp_body_first(chunk_ref, out_ref):
    out_ref[...] = chunk_ref[...].T


def _tp_body(chunk_ref, _buf_ref, out_ref):
    out_ref[...] = chunk_ref[...].T


def _make_transpose(d0, dc, aliased):
    grid = dc // _DBLK
    blk0 = d0 // _DBLK
    in_specs = [pl.BlockSpec((_B, _DBLK), lambda i: (0, i))]
    if aliased:
        in_specs.append(pl.BlockSpec(memory_space=pl.ANY))
    out_spec = pl.BlockSpec((_DBLK, _B), lambda i: (blk0 + i, 0))
    return pl.pallas_call(
        _tp_body if aliased else _tp_body_first,
        grid=(grid,),
        in_specs=in_specs,
        out_specs=out_spec,
        out_shape=jax.ShapeDtypeStruct((_D, _B), jnp.float32),
        input_output_aliases={1: 0} if aliased else {},
    )


_offsets = [sum(_CHUNKS[:k]) for k in range(len(_CHUNKS))]
_chunk_kernels = [_make_chunk_kernel(_offsets[k], _CHUNKS[k])
                  for k in range(len(_CHUNKS))]
_transpose_kernels = [_make_transpose(_offsets[k], _CHUNKS[k], aliased=(k > 0))
                      for k in range(len(_CHUNKS))]


def kernel(x, table):
    idx = x.astype(jnp.int32).reshape(_NW, _NCH, _CH)
    buf = None
    for k in range(len(_CHUNKS)):
        ok = _chunk_kernels[k](idx, table)
        buf = _transpose_kernels[k](ok) if k == 0 else _transpose_kernels[k](ok, buf)
    out_t = buf.reshape(4, 64, 64, _B)
    return out_t.transpose(3, 0, 1, 2)
